# baseline (speedup 1.0000x reference)
import functools

import jax
import jax.numpy as jnp
from jax import lax
from jax.experimental import pallas as pl
from jax.experimental.pallas import tpu as pltpu
from jax.experimental.pallas import tpu_sc as plsc

_V = 1000000
_D = 32
_B = 16384

_NC = 2
_NS = 16
_NW = _NC * _NS
_LANES = 16

_BPW = _B // _NW
_CHUNK = 128
_NCHUNK = _BPW // _CHUNK


def _body(sid_hbm, table_hbm, out_hbm, idx_v, rows_v, sem):
    wid = lax.axis_index("s") * _NC + lax.axis_index("c")
    base = wid * _BPW

    for c in range(_NCHUNK):
        pltpu.sync_copy(sid_hbm.at[pl.ds(base + c * _CHUNK, _CHUNK)],
                        idx_v.at[c])

    lo = jnp.zeros((_LANES,), jnp.int32)
    hi = jnp.full((_LANES,), _V - 1, jnp.int32)
    for c in range(_NCHUNK):
        for i in range(_CHUNK // _LANES):
            s = pl.ds(i * _LANES, _LANES)
            idx_v[c, s] = jnp.minimum(jnp.maximum(idx_v[c, s], lo), hi)

    copies = [
        pltpu.async_copy(table_hbm.at[idx_v.at[c]], rows_v.at[c], sem)
        for c in range(_NCHUNK)
    ]
    for cp in copies:
        cp.wait()

    for c in range(_NCHUNK):
        pltpu.sync_copy(rows_v.at[c],
                        out_hbm.at[pl.ds(base + c * _CHUNK, _CHUNK)])


@jax.jit
def _embed(symbol_id, table):
    mesh = plsc.VectorSubcoreMesh(core_axis_name="c", subcore_axis_name="s")
    k = functools.partial(
        pl.kernel,
        mesh=mesh,
        out_type=jax.ShapeDtypeStruct((_B, _D), jnp.float32),
        scratch_types=[
            pltpu.VMEM((_NCHUNK, _CHUNK), jnp.int32),
            pltpu.VMEM((_NCHUNK, _CHUNK, _D), jnp.float32),
            pltpu.SemaphoreType.DMA,
        ],
        compiler_params=pltpu.CompilerParams(use_tc_tiling_on_sc=False),
    )(_body)
    return k(symbol_id, table)


def kernel(symbol_id, table):
    sid = symbol_id.astype(jnp.int32)
    return _embed(sid, table)
